# Initial kernel scaffold; baseline (speedup 1.0000x reference)
import jax
import jax.numpy as jnp
from jax.experimental import pallas as pl


def kernel(target_indices, context_indices, negative_indices, target_table, context_table):
    raise NotImplementedError("write your pallas kernel here")


# speedup vs baseline: 4.5596x; 4.5596x over previous
import functools

import jax
import jax.numpy as jnp
from jax import lax
from jax.experimental import pallas as pl
from jax.experimental.pallas import tpu as pltpu
from jax.experimental.pallas import tpu_sc as plsc

D = 64
K = 20
BLK = 128
NV = D // 16


def _dot_rows(a_ref, b_ref, i):
    acc = None
    for j in range(NV):
        av = a_ref[i, pl.ds(j * 16, 16)]
        bv = b_ref[i, pl.ds(j * 16, 16)]
        p = av * bv
        acc = p if acc is None else acc + p
    return jnp.sum(acc)


def _score_loop(t_rows, b_rows, out_ref, out_idx):
    lanes = lax.iota(jnp.int32, 16)

    def body(i, sv):
        s = _dot_rows(t_rows, b_rows, i)
        li = i & 15
        sv = jnp.where(lanes == li, s, sv)

        @pl.when(li == 15)
        def _():
            out_ref[out_idx + (pl.ds(i - 15, 16),)] = sv

        return sv

    lax.fori_loop(0, BLK, body, jnp.zeros((16,), jnp.float32))


@functools.lru_cache(maxsize=None)
def _build_sc_kernel(B, NC, NS):
    NW = NC * NS
    per_w = B // NW
    nblk = per_w // BLK
    nb = B // BLK
    mesh = plsc.VectorSubcoreMesh(core_axis_name="c", subcore_axis_name="s")

    @functools.partial(
        pl.kernel,
        mesh=mesh,
        compiler_params=pltpu.CompilerParams(
            needs_layout_passes=False, use_tc_tiling_on_sc=False),
        out_type=[
            jax.ShapeDtypeStruct((B,), jnp.float32),
            jax.ShapeDtypeStruct((nb, K, BLK), jnp.float32),
        ],
        scratch_types=[
            pltpu.VMEM((nblk, BLK), jnp.int32),
            pltpu.VMEM((nblk, BLK), jnp.int32),
            pltpu.VMEM((nblk, K, BLK), jnp.int32),
            pltpu.VMEM((BLK, D), jnp.float32),
            pltpu.VMEM((BLK, D), jnp.float32),
            pltpu.VMEM((BLK, D), jnp.float32),
            pltpu.VMEM((BLK,), jnp.float32),
            pltpu.VMEM((K, BLK), jnp.float32),
            pltpu.SemaphoreType.DMA,
        ],
    )
    def sc_kernel(t_idx_hbm, c_idx_hbm, n_idx_hbm, t_tab, c_tab,
                  pos_hbm, neg_hbm,
                  t_idx_v, c_idx_v, n_idx_v, t_rows, c_rows, n_rows,
                  pos_v, neg_v, sem):
        wid = lax.axis_index("s") * NC + lax.axis_index("c")
        r0 = wid * nblk
        pltpu.sync_copy(t_idx_hbm.at[pl.ds(r0, nblk)], t_idx_v)
        pltpu.sync_copy(c_idx_hbm.at[pl.ds(r0, nblk)], c_idx_v)
        pltpu.sync_copy(n_idx_hbm.at[pl.ds(r0, nblk)], n_idx_v)

        for ib in range(nblk):
            pltpu.async_copy(t_tab.at[t_idx_v.at[ib]], t_rows, sem).wait()
            pltpu.async_copy(c_tab.at[c_idx_v.at[ib]], c_rows, sem).wait()
            _score_loop(t_rows, c_rows, pos_v, ())

            for kk in range(K):
                pltpu.async_copy(c_tab.at[n_idx_v.at[ib, kk]], n_rows,
                                 sem).wait()
                _score_loop(t_rows, n_rows, neg_v, (kk,))

            item0 = wid * per_w + ib * BLK
            pltpu.sync_copy(pos_v, pos_hbm.at[pl.ds(item0, BLK)])
            pltpu.sync_copy(neg_v, neg_hbm.at[r0 + ib])

    return sc_kernel


def kernel(target_indices, context_indices, negative_indices,
           target_table, context_table):
    B = target_indices.shape[0]
    info = plsc.get_sparse_core_info()
    nb = B // BLK
    t_idx = target_indices.reshape(nb, BLK)
    c_idx = context_indices.reshape(nb, BLK)
    n_idx = negative_indices.reshape(nb, BLK, K).transpose(0, 2, 1)
    run = _build_sc_kernel(B, info.num_cores, info.num_subcores)
    pos, neg = run(t_idx, c_idx, n_idx, target_table, context_table)
    return pos.reshape(B, 1), neg.transpose(0, 2, 1).reshape(B, K)
